# baseline (speedup 1.0000x reference)
import functools
import jax
import jax.numpy as jnp
from jax import lax
from jax.experimental import pallas as pl
from jax.experimental.pallas import tpu as pltpu
from jax.experimental.pallas import tpu_sc as plsc

IOU_THRS = (0.5, 0.75)
BIG = 1e9
BIG_I = 1 << 30

N_PAD = 20480
N_WORDS = N_PAD // 32
TP_CAP = 2048
LBL_BLK = 8
ROW_BATCH = 16
CHUNKS = N_WORDS // 16


def _mask_body(pb_ref, pe_ref, lbl_ref, b5_ref, b7_ref):
    tb = lbl_ref[:, 0:1]
    te = lbl_ref[:, 1:2]
    acc5 = jnp.zeros((LBL_BLK, N_WORDS), jnp.int32)
    acc7 = jnp.zeros((LBL_BLK, N_WORDS), jnp.int32)
    for b in range(32):
        pb = pb_ref[b:b + 1, :]
        pe = pe_ref[b:b + 1, :]
        inner = jnp.maximum(jnp.minimum(pe, te) - jnp.maximum(pb, tb), 0.0)
        outer = jnp.maximum(pe, te) - jnp.minimum(pb, tb)
        m5 = (inner > jnp.float32(IOU_THRS[0]) * outer).astype(jnp.int32)
        m7 = (inner > jnp.float32(IOU_THRS[1]) * outer).astype(jnp.int32)
        acc5 = acc5 | (m5 << b)
        acc7 = acc7 | (m7 << b)
    b5_ref[...] = acc5
    b7_ref[...] = acc7


def _greedy_body(bits_hbm, tpi_hbm, rowbuf, used_v, tpi_v, tmp_v, dma_sem,
                 *, n_lab):
    t = lax.axis_index("c")
    active = lax.axis_index("s") == 0
    n_batch = n_lab // ROW_BATCH

    @pl.when(active)
    def _():
        lane = lax.iota(jnp.int32, 16)

        def initw(w, _):
            used_v[pl.ds(w * 16, 16)] = jnp.zeros((16,), jnp.int32)
            return 0
        lax.fori_loop(0, CHUNKS, initw, 0)

        def initt(k, _):
            tpi_v[pl.ds(k * 16, 16)] = jnp.full((16,), -1, jnp.int32)
            return 0
        lax.fori_loop(0, (TP_CAP + 16) // 16, initt, 0)

        def treemin(vec):
            tmp_v[pl.ds(0, 16)] = vec
            tmp_v[pl.ds(16, 16)] = vec
            for s in (8, 4, 2, 1):
                nv = jnp.minimum(tmp_v[pl.ds(0, 16)], tmp_v[pl.ds(s, 16)])
                tmp_v[pl.ds(0, 16)] = nv
                tmp_v[pl.ds(16, 16)] = nv
            return tmp_v[pl.ds(0, 16)][0]

        def batch(g, cnt):
            pltpu.async_copy(
                bits_hbm.at[t, pl.ds(g * ROW_BATCH, ROW_BATCH), :],
                rowbuf, dma_sem).wait()
            for r in range(ROW_BATCH):
                def wbody(w, wmv):
                    m = rowbuf[r, pl.ds(w * 16, 16)]
                    u = used_v[pl.ds(w * 16, 16)]
                    free = m & ~u
                    cand = jnp.where(free != 0, w * 16 + lane, BIG_I)
                    return jnp.minimum(wmv, cand)

                wmv = lax.fori_loop(0, CHUNKS, wbody,
                                    jnp.full((16,), BIG_I, jnp.int32))
                wmin = treemin(wmv)
                found = wmin < N_WORDS

                @pl.when(found)
                def _():
                    base = pl.multiple_of((wmin >> 4) << 4, 16)
                    m = rowbuf[r, pl.ds(base, 16)]
                    u = used_v[pl.ds(base, 16)]
                    free = m & ~u
                    lowbit = free & (0 - free)
                    sel = (base + lane) == wmin
                    used_v[pl.ds(base, 16)] = u | jnp.where(sel, lowbit, 0)
                    e = jnp.zeros((16,), jnp.int32)
                    for k, msk in ((1, -1431655766), (2, -858993460),
                                   (4, -252645136), (8, -16711936),
                                   (16, -65536)):
                        e = e + jnp.where((lowbit & jnp.int32(msk)) != 0, k, 0)
                    jc = ((base + lane) << 5) + e
                    jsplat = treemin(jnp.where(sel, jc, BIG_I))
                    cb = pl.multiple_of((cnt >> 4) << 4, 16)
                    tslot = tpi_v[pl.ds(cb, 16)]
                    tpi_v[pl.ds(cb, 16)] = jnp.where(lane == cnt - cb,
                                                     jsplat, tslot)

                cnt = jnp.where(found, cnt + 1, cnt)
            return cnt

        lax.fori_loop(0, n_batch, batch, jnp.int32(0))
        pltpu.sync_copy(tpi_v.at[pl.ds(0, TP_CAP)], tpi_hbm.at[t])


TP_ROWS = 16
TP_LANES = 128
CH = 256


def _ap_reduce(ti_ref, tc_ref, r_ref, p_ref, conff_ref, *, n_lab, n_flat):
    neg1 = jnp.float32(-1.0)
    ti3 = ti_ref[...].reshape(TP_ROWS, TP_LANES, 1)
    n_ch = n_flat // CH

    def ebody(c, acc):
        cf = conff_ref[:, pl.ds(c * CH, CH)].reshape(1, 1, CH)
        ji = lax.broadcasted_iota(jnp.int32, (1, 1, CH), 2) + c * CH
        return acc + jnp.sum(jnp.where(ji == ti3, cf, 0.0), axis=2)

    tc = lax.fori_loop(0, n_ch, ebody,
                       jnp.zeros((TP_ROWS, TP_LANES), dtype=jnp.float32))
    tc_ref[...] = tc
    tc3 = tc.reshape(TP_ROWS, TP_LANES, 1)

    def rbody(c, acc):
        cf = conff_ref[:, pl.ds(c * CH, CH)].reshape(1, 1, CH)
        ji = lax.broadcasted_iota(jnp.int32, (1, 1, CH), 2) + c * CH
        gt = cf > tc3
        tie = (cf == tc3) & (ji < ti3)
        return acc + jnp.sum((gt | tie).astype(jnp.float32), axis=2)

    r = lax.fori_loop(0, n_ch, rbody,
                      jnp.ones((TP_ROWS, TP_LANES), dtype=jnp.float32))
    r_ref[...] = r
    r3 = r.reshape(TP_ROWS, TP_LANES, 1)

    def cbody(q, acc):
        rq = r_ref[pl.ds(q, 1), :].reshape(1, 1, TP_LANES)
        vq = (ti_ref[pl.ds(q, 1), :] >= 0).reshape(1, 1, TP_LANES)
        return acc + jnp.sum((vq & (rq < r3)).astype(jnp.float32), axis=2)

    c = lax.fori_loop(0, TP_ROWS, cbody,
                      jnp.ones((TP_ROWS, TP_LANES), dtype=jnp.float32))
    p_ref[...] = c / r

    def mbody(q, acc):
        rq = r_ref[pl.ds(q, 1), :].reshape(1, 1, TP_LANES)
        vq = (ti_ref[pl.ds(q, 1), :] >= 0).reshape(1, 1, TP_LANES)
        pq = p_ref[pl.ds(q, 1), :].reshape(1, 1, TP_LANES)
        return jnp.maximum(acc, jnp.max(jnp.where(vq & (rq >= r3), pq, neg1),
                                        axis=2))

    M = lax.fori_loop(0, TP_ROWS, mbody,
                      jnp.full((TP_ROWS, TP_LANES), neg1, dtype=jnp.float32))
    valid = ti_ref[...] >= 0
    take = valid & (r >= jnp.float32(2.0))
    return jnp.sum(jnp.where(take, M, jnp.float32(0.0))) / jnp.float32(n_lab)


def _apred_body(tpi_ref, conff_ref, out_ref,
                tc5_ref, r5_ref, p5_ref, tc7_ref, r7_ref, p7_ref,
                *, n_lab, n_flat):
    ap5 = _ap_reduce(tpi_ref.at[0], tc5_ref, r5_ref, p5_ref, conff_ref,
                     n_lab=n_lab, n_flat=n_flat)
    ap7 = _ap_reduce(tpi_ref.at[1], tc7_ref, r7_ref, p7_ref, conff_ref,
                     n_lab=n_lab, n_flat=n_flat)

    lanes = lax.broadcasted_iota(jnp.int32, (8, 128), 1)
    sub = lax.broadcasted_iota(jnp.int32, (8, 128), 0)
    out_ref[...] = jnp.where((sub == 0) & (lanes == 0), ap5,
                             jnp.where((sub == 0) & (lanes == 1), ap7,
                                       jnp.float32(0.0)))


def kernel(proposals, labels):
    n_prop = proposals.shape[0]
    n_lab = labels.shape[0]
    conf = proposals[:, 0]
    pbv = proposals[:, 1]
    pev = proposals[:, 2]
    padn = N_PAD - n_prop
    conf_p = jnp.pad(conf, (0, padn), constant_values=-1.0)
    pb_p = jnp.pad(pbv, (0, padn), constant_values=-1.0)
    pe_p = jnp.pad(pev, (0, padn), constant_values=-1.0)
    pb32 = pb_p.reshape(N_WORDS, 32).T
    pe32 = pe_p.reshape(N_WORDS, 32).T

    grid = n_lab // LBL_BLK
    b5, b7 = pl.pallas_call(
        _mask_body,
        grid=(grid,),
        in_specs=[
            pl.BlockSpec((32, N_WORDS), lambda i: (0, 0)),
            pl.BlockSpec((32, N_WORDS), lambda i: (0, 0)),
            pl.BlockSpec((LBL_BLK, 2), lambda i: (i, 0)),
        ],
        out_specs=[
            pl.BlockSpec((LBL_BLK, N_WORDS), lambda i: (i, 0)),
            pl.BlockSpec((LBL_BLK, N_WORDS), lambda i: (i, 0)),
        ],
        out_shape=[
            jax.ShapeDtypeStruct((n_lab, N_WORDS), jnp.int32),
            jax.ShapeDtypeStruct((n_lab, N_WORDS), jnp.int32),
        ],
    )(pb32, pe32, labels)
    bits = jnp.stack([b5, b7])

    mesh = plsc.VectorSubcoreMesh(core_axis_name="c", subcore_axis_name="s",
                                  num_cores=2)
    tpi = pl.kernel(
        functools.partial(_greedy_body, n_lab=n_lab),
        out_type=jax.ShapeDtypeStruct((2, TP_CAP), jnp.int32),
        mesh=mesh,
        scratch_types=[
            pltpu.VMEM((ROW_BATCH, N_WORDS), jnp.int32),
            pltpu.VMEM((N_WORDS,), jnp.int32),
            pltpu.VMEM((TP_CAP + 16,), jnp.int32),
            pltpu.VMEM((32,), jnp.int32),
            pltpu.SemaphoreType.DMA,
        ],
    )(bits)

    n_flat = N_PAD
    out2d = pl.pallas_call(
        functools.partial(_apred_body, n_lab=n_lab, n_flat=n_flat),
        in_specs=[
            pl.BlockSpec(memory_space=pltpu.VMEM),
            pl.BlockSpec(memory_space=pltpu.VMEM),
        ],
        out_specs=pl.BlockSpec(memory_space=pltpu.VMEM),
        out_shape=jax.ShapeDtypeStruct((8, 128), jnp.float32),
        scratch_shapes=[
            pltpu.VMEM((TP_ROWS, TP_LANES), jnp.float32),
            pltpu.VMEM((TP_ROWS, TP_LANES), jnp.float32),
            pltpu.VMEM((TP_ROWS, TP_LANES), jnp.float32),
            pltpu.VMEM((TP_ROWS, TP_LANES), jnp.float32),
            pltpu.VMEM((TP_ROWS, TP_LANES), jnp.float32),
            pltpu.VMEM((TP_ROWS, TP_LANES), jnp.float32),
        ],
    )(tpi.reshape(2, TP_ROWS, TP_LANES), conf_p[None, :])
    return out2d[0, :2]
